# Initial kernel scaffold; baseline (speedup 1.0000x reference)
import jax
import jax.numpy as jnp
from jax.experimental import pallas as pl


def kernel(h, edge_index):
    raise NotImplementedError("write your pallas kernel here")


# speedup vs baseline: 3.2357x; 3.2357x over previous
import functools

import jax
import jax.numpy as jnp
from jax import lax
from jax.experimental import pallas as pl
from jax.experimental.pallas import tpu as pltpu
from jax.experimental.pallas import tpu_sc as plsc

D = 128
L = 16
NC = 2
NS = 16
NW = NC * NS
C = 80


@functools.partial(jax.jit, static_argnames=("E", "N"))
def _score(h, src, dst, *, E, N):
    EW = E // NW
    NCH = EW // C
    G = C // L

    mesh = plsc.VectorSubcoreMesh(
        core_axis_name="c", subcore_axis_name="s", num_cores=NC,
        num_subcores=NS)

    @functools.partial(
        pl.kernel,
        out_type=jax.ShapeDtypeStruct((E,), jnp.float32),
        mesh=mesh,
        scratch_types=[
            pltpu.VMEM((C,), jnp.int32),
            pltpu.VMEM((C,), jnp.int32),
            pltpu.VMEM((C, D), jnp.float32),
            pltpu.VMEM((C, D), jnp.float32),
            pltpu.VMEM((C,), jnp.float32),
            pltpu.SemaphoreType.DMA,
            pltpu.SemaphoreType.DMA,
        ],
        compiler_params=pltpu.CompilerParams(needs_layout_passes=False),
    )
    def k(h_hbm, src_hbm, dst_hbm, out_hbm,
          sidx, didx, srows, drows, scores, sem0, sem1):
        wid = lax.axis_index("s") * NC + lax.axis_index("c")
        base_w = wid * EW
        lane = lax.iota(jnp.int32, L)
        last_lane = lane == (L - 1)

        def chunk_body(i, carry):
            base = base_w + i * C
            pltpu.sync_copy(src_hbm.at[pl.ds(base, C)], sidx)
            pltpu.sync_copy(dst_hbm.at[pl.ds(base, C)], didx)
            cp0 = pltpu.async_copy(h_hbm.at[sidx], srows, sem0)
            cp1 = pltpu.async_copy(h_hbm.at[didx], drows, sem1)
            cp0.wait()
            cp1.wait()

            def edge_body(e, carry2):
                acc = srows[e, pl.ds(0, L)] * drows[e, pl.ds(0, L)]
                for j in range(1, D // L):
                    acc = acc + (srows[e, pl.ds(j * L, L)] *
                                 drows[e, pl.ds(j * L, L)])
                csum = lax.cumsum(acc)
                plsc.store_scatter(scores, [jnp.broadcast_to(e, (L,))],
                                   csum, mask=last_lane)
                return carry2

            lax.fori_loop(0, C, edge_body, 0, unroll=False)
            pltpu.sync_copy(scores, out_hbm.at[pl.ds(base, C)])
            return carry

        lax.fori_loop(0, NCH, chunk_body, 0, unroll=False)

    return k(h, src, dst)


def kernel(h, edge_index):
    N, d = h.shape
    E = edge_index.shape[1]
    src = edge_index[0].astype(jnp.int32)
    dst = edge_index[1].astype(jnp.int32)
    out = _score(h, src, dst, E=E, N=N)
    return out.reshape(E, 1)


# speedup vs baseline: 6.7524x; 2.0868x over previous
import functools

import jax
import jax.numpy as jnp
from jax import lax
from jax.experimental import pallas as pl
from jax.experimental.pallas import tpu as pltpu
from jax.experimental.pallas import tpu_sc as plsc

D = 128
L = 16
NC = 2
NS = 16
NW = NC * NS
C = 80
NB = 2


@functools.partial(jax.jit, static_argnames=("E", "N"))
def _score(h, src, dst, *, E, N):
    EW = E // NW
    NCH = EW // C

    mesh = plsc.VectorSubcoreMesh(
        core_axis_name="c", subcore_axis_name="s", num_cores=NC,
        num_subcores=NS)

    @functools.partial(
        pl.kernel,
        out_type=jax.ShapeDtypeStruct((E,), jnp.float32),
        mesh=mesh,
        scratch_types=[
            pltpu.VMEM((EW,), jnp.int32),
            pltpu.VMEM((EW,), jnp.int32),
            pltpu.VMEM((NB, C, D), jnp.float32),
            pltpu.VMEM((NB, C, D), jnp.float32),
            pltpu.VMEM((EW,), jnp.float32),
            pltpu.SemaphoreType.DMA,
            pltpu.SemaphoreType.DMA,
            pltpu.SemaphoreType.DMA,
            pltpu.SemaphoreType.DMA,
        ],
        compiler_params=pltpu.CompilerParams(needs_layout_passes=False),
    )
    def k(h_hbm, src_hbm, dst_hbm, out_hbm,
          sidx, didx, srows, drows, scores,
          sem_s0, sem_s1, sem_d0, sem_d1):
        wid = lax.axis_index("s") * NC + lax.axis_index("c")
        base_w = wid * EW
        lane = lax.iota(jnp.int32, L)
        last_lane = lane == (L - 1)
        sems_s = (sem_s0, sem_s1)
        sems_d = (sem_d0, sem_d1)

        pltpu.sync_copy(src_hbm.at[pl.ds(base_w, EW)], sidx)
        pltpu.sync_copy(dst_hbm.at[pl.ds(base_w, EW)], didx)

        def issue(chunk, b):
            pltpu.async_copy(
                h_hbm.at[sidx.at[pl.ds(chunk * C, C)]], srows.at[b],
                sems_s[b])
            pltpu.async_copy(
                h_hbm.at[didx.at[pl.ds(chunk * C, C)]], drows.at[b],
                sems_d[b])

        def wait(b):
            pltpu.make_async_copy(
                h_hbm.at[sidx.at[pl.ds(0, C)]], srows.at[b],
                sems_s[b]).wait()
            pltpu.make_async_copy(
                h_hbm.at[didx.at[pl.ds(0, C)]], drows.at[b],
                sems_d[b]).wait()

        def compute(chunk, b):
            wait(b)
            sr = srows.at[b]
            dr = drows.at[b]
            ebase = chunk * C

            @pl.loop(0, C, unroll=2)
            def edge_body(e):
                acc = sr[e, pl.ds(0, L)] * dr[e, pl.ds(0, L)]
                for j in range(1, D // L):
                    acc = acc + (sr[e, pl.ds(j * L, L)] *
                                 dr[e, pl.ds(j * L, L)])
                csum = lax.cumsum(acc)
                plsc.store_scatter(scores, [jnp.broadcast_to(ebase + e, (L,))],
                                   csum, mask=last_lane)

        for b in range(NB):
            issue(b, b)

        NMAIN = NCH - NCH % NB

        @pl.loop(0, NMAIN, step=NB)
        def chunk_loop(i):
            for b in range(NB):
                chunk = i + b
                compute(chunk, b)

                @pl.when(chunk + NB < NCH)
                def _():
                    issue(chunk + NB, b)

        for t in range(NCH % NB):
            compute(NMAIN + t, t)

        pltpu.sync_copy(scores, out_hbm.at[pl.ds(base_w, EW)])

    return k(h, src, dst)


def kernel(h, edge_index):
    N, d = h.shape
    E = edge_index.shape[1]
    src = edge_index[0].astype(jnp.int32)
    dst = edge_index[1].astype(jnp.int32)
    out = _score(h, src, dst, E=E, N=N)
    return out.reshape(E, 1)
